# baseline (speedup 1.0000x reference)
import functools

import jax
import jax.numpy as jnp
from jax import lax
from jax.experimental import pallas as pl
from jax.experimental.pallas import tpu as pltpu
from jax.experimental.pallas import tpu_sc as plsc

N_CENTER = 50000
N_NEIGH = 4
F = 32
OUTW = 3 + 3 * F
PAIRS = [(j, k) for j in range(N_NEIGH) for k in range(N_NEIGH) if j != k]
NP_ = len(PAIRS)
ROW = NP_ * OUTW

NC, NS, L = 2, 16, 16
NW = NC * NS
G = L
N_GROUPS = N_CENTER // G
GPW_HI = -(-N_GROUPS // NW)
CPW = GPW_HI * G
REM = N_GROUPS - NW * (GPW_HI - 1)
GBUF = G * ROW
NJ = CPW * N_NEIGH


def _safe_sqrt(sq):
    i = plsc.bitcast(sq, jnp.int32)
    y = plsc.bitcast(jnp.int32(0x5F3759DF) - (i >> 1), jnp.float32)
    for _ in range(3):
        y = y * (1.5 - 0.5 * sq * y * y)
    return jnp.where(sq > 0.0, sq * y, 0.0)


def _tec_body(jflat, iidx, dflat, xyzflat, alflat, emb, ang,
              emb_v, ji_v, ii_v, d_v, jx_v, jy_v, jz_v, jt_v, it_v,
              xv_v, yv_v, zv_v, tj_v, ti_v, rd_v, out_v, sem_in, sem_out):
    wid = lax.axis_index("s") * NC + lax.axis_index("c")
    g0 = wid * (GPW_HI - 1) + jnp.minimum(wid, REM)
    ng = (GPW_HI - 1) + (wid < REM).astype(jnp.int32)
    base_c = g0 * G
    in_base = jnp.minimum(base_c, N_CENTER - CPW)
    off = base_c - in_base

    cps = [
        pltpu.async_copy(dflat.at[pl.ds(in_base * N_NEIGH, NJ)], d_v, sem_in),
        pltpu.async_copy(jflat.at[pl.ds(in_base * N_NEIGH, NJ)], ji_v, sem_in),
        pltpu.async_copy(iidx.at[pl.ds(in_base, CPW)], ii_v, sem_in),
        pltpu.async_copy(emb, emb_v, sem_in),
    ]
    for c in cps:
        c.wait()

    @pl.loop(0, NJ // L)
    def _mkidx(b):
        sl = pl.ds(b * L, L)
        ji = ji_v[sl]
        j3 = ji * 3
        jx_v[sl] = j3
        jy_v[sl] = j3 + 1
        jz_v[sl] = j3 + 2
        jt_v[sl] = ji * 2 + 1

    @pl.loop(0, CPW // L)
    def _mkidx2(b):
        sl = pl.ds(b * L, L)
        it_v[sl] = ii_v[sl] * 2 + 1

    gcs = []
    for c in range(NJ // 128):
        sl = pl.ds(c * 128, 128)
        gcs.append(pltpu.async_copy(xyzflat.at[jx_v.at[sl]], xv_v.at[sl],
                                    sem_in))
        gcs.append(pltpu.async_copy(xyzflat.at[jy_v.at[sl]], yv_v.at[sl],
                                    sem_in))
        gcs.append(pltpu.async_copy(xyzflat.at[jz_v.at[sl]], zv_v.at[sl],
                                    sem_in))
        gcs.append(pltpu.async_copy(alflat.at[jt_v.at[sl]], tj_v.at[sl],
                                    sem_in))
    for c in range(CPW // 112):
        sl = pl.ds(c * 112, 112)
        gcs.append(pltpu.async_copy(alflat.at[it_v.at[sl]], ti_v.at[sl],
                                    sem_in))
    for c in gcs:
        c.wait()

    lane = lax.iota(jnp.int32, L)

    @pl.loop(0, ng)
    def _group(g):
        bufrow = (g % 2) * G
        @pl.when(g >= 2)
        def _drain():
            pltpu.make_async_copy(out_v.at[pl.ds(bufrow, G)],
                                  ang.at[pl.ds(base_c, G)], sem_out).wait()

        lg = g * G + off
        lc = lane + lg
        lj = lc * N_NEIGH
        d = [plsc.load_gather(d_v, [lj + j]) for j in range(N_NEIGH)]
        for j in range(N_NEIGH):
            plsc.store_scatter(rd_v, [lane * N_NEIGH + j], 1.0 / d[j])
        x = [[plsc.load_gather(cv, [lj + j]) for cv in (xv_v, yv_v, zv_v)]
             for j in range(N_NEIGH)]

        rrow = bufrow + lane
        tjk = {}
        for (j, k) in PAIRS:
            if j < k:
                dx = x[j][0] - x[k][0]
                dy = x[j][1] - x[k][1]
                dz = x[j][2] - x[k][2]
                sq = dx * dx + dy * dy + dz * dz
                tjk[(j, k)] = _safe_sqrt(sq)
            else:
                tjk[(j, k)] = tjk[(k, j)]

        for p, (j, k) in enumerate(PAIRS):
            pv = jnp.full((L,), p, jnp.int32)
            mind = jnp.minimum(d[j], d[k])
            maxd = jnp.maximum(d[j], d[k])
            tn = (tjk[(j, k)] - maxd + mind) / (2.0 * mind)
            plsc.store_scatter(out_v, [rrow, pv, jnp.full((L,), 0, jnp.int32)],
                               d[j])
            plsc.store_scatter(out_v, [rrow, pv, jnp.full((L,), 1, jnp.int32)],
                               d[k])
            plsc.store_scatter(out_v, [rrow, pv, jnp.full((L,), 2, jnp.int32)],
                               tn)

        for c in range(G):
            lcc = lg + c
            tiv = ti_v[pl.ds(lcc, L)]
            tjv = tj_v[pl.ds(lcc * N_NEIGH, L)]
            rdv = rd_v[pl.ds(c * N_NEIGH, L)]
            ei_a = tiv[0] * F
            e_i = [emb_v[pl.ds(ei_a, L)], emb_v[pl.ds(ei_a + L, L)]]
            ejs = []
            for j in range(N_NEIGH):
                ej_a = tjv[j] * F
                r = rdv[j]
                ejs.append([emb_v[pl.ds(ej_a, L)] * r,
                            emb_v[pl.ds(ej_a + L, L)] * r])
            row = bufrow + c
            for p, (j, k) in enumerate(PAIRS):
                out_v[row, p, pl.ds(3, L)] = e_i[0]
                out_v[row, p, pl.ds(3 + L, L)] = e_i[1]
                out_v[row, p, pl.ds(3 + F, L)] = ejs[j][0]
                out_v[row, p, pl.ds(3 + F + L, L)] = ejs[j][1]
                out_v[row, p, pl.ds(3 + 2 * F, L)] = ejs[k][0]
                out_v[row, p, pl.ds(3 + 2 * F + L, L)] = ejs[k][1]

        pltpu.async_copy(out_v.at[pl.ds(bufrow, G)],
                         ang.at[pl.ds(base_c + g * G, G)], sem_out)

    for _ in range(2):
        pltpu.make_async_copy(out_v.at[pl.ds(0, G)],
                              ang.at[pl.ds(base_c, G)], sem_out).wait()


@jax.jit
def _run(jflat, iidx, dflat, xyzflat, alflat, emb_flat):
    mesh = plsc.VectorSubcoreMesh(core_axis_name="c", subcore_axis_name="s",
                                  num_cores=NC, num_subcores=NS)
    kern = functools.partial(
        pl.kernel,
        out_type=jax.ShapeDtypeStruct((N_CENTER, NP_, OUTW), jnp.float32),
        mesh=mesh,
        compiler_params=pltpu.CompilerParams(needs_layout_passes=False,
                                             use_tc_tiling_on_sc=False),
        scratch_types=[
            pltpu.VMEM((100 * F,), jnp.float32),
            pltpu.VMEM((NJ,), jnp.int32),
            pltpu.VMEM((CPW,), jnp.int32),
            pltpu.VMEM((NJ,), jnp.float32),
            pltpu.VMEM((NJ,), jnp.int32),
            pltpu.VMEM((NJ,), jnp.int32),
            pltpu.VMEM((NJ,), jnp.int32),
            pltpu.VMEM((NJ,), jnp.int32),
            pltpu.VMEM((CPW,), jnp.int32),
            pltpu.VMEM((NJ,), jnp.float32),
            pltpu.VMEM((NJ,), jnp.float32),
            pltpu.VMEM((NJ,), jnp.float32),
            pltpu.VMEM((NJ + L,), jnp.int32),
            pltpu.VMEM((CPW + L,), jnp.int32),
            pltpu.VMEM((N_NEIGH * L + L,), jnp.float32),
            pltpu.VMEM((2 * G, NP_, OUTW), jnp.float32),
            pltpu.SemaphoreType.DMA,
            pltpu.SemaphoreType.DMA,
        ],
    )(_tec_body)
    return kern(jflat, iidx, dflat, xyzflat, alflat, emb_flat)


def kernel(nNeigh, atom_i_idx, atom_j_idx, dist_ij, atoms_xyz, atoms_long,
           embed_table):
    jflat = atom_j_idx.reshape(-1).astype(jnp.int32)
    iidx = atom_i_idx.astype(jnp.int32)
    dflat = dist_ij.reshape(-1)
    emb_flat = embed_table.reshape(-1)
    xyzflat = atoms_xyz.reshape(-1)
    alflat = atoms_long.reshape(-1).astype(jnp.int32)
    ang = _run(jflat, iidx, dflat, xyzflat, alflat, emb_flat)
    return atom_i_idx.reshape(-1), ang


# speedup vs baseline: 1.0669x; 1.0669x over previous
import functools

import jax
import jax.numpy as jnp
from jax import lax
from jax.experimental import pallas as pl
from jax.experimental.pallas import tpu as pltpu
from jax.experimental.pallas import tpu_sc as plsc

N_CENTER = 50000
N_NEIGH = 4
F = 32
OUTW = 3 + 3 * F
PAIRS = [(j, k) for j in range(N_NEIGH) for k in range(N_NEIGH) if j != k]
NP_ = len(PAIRS)
ROW = NP_ * OUTW

NC, NS, L = 2, 16, 16
NW = NC * NS
G = L
N_GROUPS = N_CENTER // G
GPW_HI = -(-N_GROUPS // NW)
CPW = GPW_HI * G
REM = N_GROUPS - NW * (GPW_HI - 1)
GBUF = G * ROW
NJ = CPW * N_NEIGH


def _safe_sqrt(sq):
    i = plsc.bitcast(sq, jnp.int32)
    y = plsc.bitcast(jnp.int32(0x5F3759DF) - (i >> 1), jnp.float32)
    for _ in range(3):
        y = y * (1.5 - 0.5 * sq * y * y)
    return jnp.where(sq > 0.0, sq * y, 0.0)


def _tec_body(jflat, iidx, dflat, xyzflat, alflat, emb, ang,
              emb_v, ji_v, ii_v, d_v, jx_v, jy_v, jz_v, jt_v, it_v,
              xv_v, yv_v, zv_v, tj_v, ti_v, rd_v, out_v, sem_in, sem_out):
    wid = lax.axis_index("s") * NC + lax.axis_index("c")
    g0 = wid * (GPW_HI - 1) + jnp.minimum(wid, REM)
    ng = (GPW_HI - 1) + (wid < REM).astype(jnp.int32)
    base_c = g0 * G
    in_base = jnp.minimum(base_c, N_CENTER - CPW)
    off = base_c - in_base

    cps = [
        pltpu.async_copy(dflat.at[pl.ds(in_base * N_NEIGH, NJ)], d_v, sem_in),
        pltpu.async_copy(jflat.at[pl.ds(in_base * N_NEIGH, NJ)], ji_v, sem_in),
        pltpu.async_copy(iidx.at[pl.ds(in_base, CPW)], ii_v, sem_in),
        pltpu.async_copy(emb, emb_v, sem_in),
    ]
    for c in cps:
        c.wait()

    @pl.loop(0, NJ // L)
    def _mkidx(b):
        sl = pl.ds(b * L, L)
        ji = ji_v[sl]
        j3 = ji * 3
        jx_v[sl] = j3
        jy_v[sl] = j3 + 1
        jz_v[sl] = j3 + 2
        jt_v[sl] = ji * 2 + 1

    @pl.loop(0, CPW // L)
    def _mkidx2(b):
        sl = pl.ds(b * L, L)
        it_v[sl] = ii_v[sl] * 2 + 1

    gcs = []
    for c in range(NJ // 128):
        sl = pl.ds(c * 128, 128)
        gcs.append(pltpu.async_copy(xyzflat.at[jx_v.at[sl]], xv_v.at[sl],
                                    sem_in))
        gcs.append(pltpu.async_copy(xyzflat.at[jy_v.at[sl]], yv_v.at[sl],
                                    sem_in))
        gcs.append(pltpu.async_copy(xyzflat.at[jz_v.at[sl]], zv_v.at[sl],
                                    sem_in))
        gcs.append(pltpu.async_copy(alflat.at[jt_v.at[sl]], tj_v.at[sl],
                                    sem_in))
    for c in range(CPW // 112):
        sl = pl.ds(c * 112, 112)
        gcs.append(pltpu.async_copy(alflat.at[it_v.at[sl]], ti_v.at[sl],
                                    sem_in))
    for c in gcs:
        c.wait()

    lane = lax.iota(jnp.int32, L)

    @pl.loop(0, ng)
    def _group(g):
        buf = (g % 2) * GBUF
        @pl.when(g >= 2)
        def _drain():
            pltpu.make_async_copy(out_v.at[pl.ds(buf, GBUF)],
                                  ang.at[pl.ds(base_c * ROW, GBUF)],
                                  sem_out).wait()

        lg = g * G + off
        lc = lane + lg
        lj = lc * N_NEIGH
        d = [plsc.load_gather(d_v, [lj + j]) for j in range(N_NEIGH)]
        for j in range(N_NEIGH):
            plsc.store_scatter(rd_v, [lane * N_NEIGH + j], 1.0 / d[j])
        x = [[plsc.load_gather(cv, [lj + j]) for cv in (xv_v, yv_v, zv_v)]
             for j in range(N_NEIGH)]

        obase = buf + lane * ROW
        tjk = {}
        for (j, k) in PAIRS:
            if j < k:
                dx = x[j][0] - x[k][0]
                dy = x[j][1] - x[k][1]
                dz = x[j][2] - x[k][2]
                sq = dx * dx + dy * dy + dz * dz
                tjk[(j, k)] = _safe_sqrt(sq)
            else:
                tjk[(j, k)] = tjk[(k, j)]

        for p, (j, k) in enumerate(PAIRS):
            c0 = obase + p * OUTW
            mind = jnp.minimum(d[j], d[k])
            maxd = jnp.maximum(d[j], d[k])
            tn = (tjk[(j, k)] - maxd + mind) / (2.0 * mind)
            plsc.store_scatter(out_v, [c0], d[j])
            plsc.store_scatter(out_v, [c0 + 1], d[k])
            plsc.store_scatter(out_v, [c0 + 2], tn)

        for c in range(G):
            lcc = lg + c
            tiv = ti_v[pl.ds(lcc, L)]
            tjv = tj_v[pl.ds(lcc * N_NEIGH, L)]
            rdv = rd_v[pl.ds(c * N_NEIGH, L)]
            ei_a = tiv[0] * F
            e_i = [emb_v[pl.ds(ei_a, L)], emb_v[pl.ds(ei_a + L, L)]]
            ejs = []
            for j in range(N_NEIGH):
                ej_a = tjv[j] * F
                r = rdv[j]
                ejs.append([emb_v[pl.ds(ej_a, L)] * r,
                            emb_v[pl.ds(ej_a + L, L)] * r])
            ob = buf + c * ROW
            for p, (j, k) in enumerate(PAIRS):
                cb = ob + p * OUTW + 3
                out_v[pl.ds(cb, L)] = e_i[0]
                out_v[pl.ds(cb + L, L)] = e_i[1]
                out_v[pl.ds(cb + F, L)] = ejs[j][0]
                out_v[pl.ds(cb + F + L, L)] = ejs[j][1]
                out_v[pl.ds(cb + 2 * F, L)] = ejs[k][0]
                out_v[pl.ds(cb + 2 * F + L, L)] = ejs[k][1]

        pltpu.async_copy(
            out_v.at[pl.ds(buf, GBUF)],
            ang.at[pl.ds((base_c + g * G) * ROW, GBUF)],
            sem_out)

    for _ in range(2):
        pltpu.make_async_copy(out_v.at[pl.ds(0, GBUF)],
                              ang.at[pl.ds(base_c * ROW, GBUF)],
                              sem_out).wait()


@jax.jit
def _run(jflat, iidx, dflat, xyzflat, alflat, emb_flat):
    mesh = plsc.VectorSubcoreMesh(core_axis_name="c", subcore_axis_name="s",
                                  num_cores=NC, num_subcores=NS)
    kern = functools.partial(
        pl.kernel,
        out_type=jax.ShapeDtypeStruct((N_CENTER * ROW,), jnp.float32),
        mesh=mesh,
        compiler_params=pltpu.CompilerParams(needs_layout_passes=False,
                                             use_tc_tiling_on_sc=False),
        scratch_types=[
            pltpu.VMEM((100 * F,), jnp.float32),
            pltpu.VMEM((NJ,), jnp.int32),
            pltpu.VMEM((CPW,), jnp.int32),
            pltpu.VMEM((NJ,), jnp.float32),
            pltpu.VMEM((NJ,), jnp.int32),
            pltpu.VMEM((NJ,), jnp.int32),
            pltpu.VMEM((NJ,), jnp.int32),
            pltpu.VMEM((NJ,), jnp.int32),
            pltpu.VMEM((CPW,), jnp.int32),
            pltpu.VMEM((NJ,), jnp.float32),
            pltpu.VMEM((NJ,), jnp.float32),
            pltpu.VMEM((NJ,), jnp.float32),
            pltpu.VMEM((NJ + L,), jnp.int32),
            pltpu.VMEM((CPW + L,), jnp.int32),
            pltpu.VMEM((N_NEIGH * L + L,), jnp.float32),
            pltpu.VMEM((2 * GBUF,), jnp.float32),
            pltpu.SemaphoreType.DMA,
            pltpu.SemaphoreType.DMA,
        ],
    )(_tec_body)
    return kern(jflat, iidx, dflat, xyzflat, alflat, emb_flat)


def kernel(nNeigh, atom_i_idx, atom_j_idx, dist_ij, atoms_xyz, atoms_long,
           embed_table):
    jflat = atom_j_idx.reshape(-1).astype(jnp.int32)
    iidx = atom_i_idx.astype(jnp.int32)
    dflat = dist_ij.reshape(-1)
    emb_flat = embed_table.reshape(-1)
    xyzflat = atoms_xyz.reshape(-1)
    alflat = atoms_long.reshape(-1).astype(jnp.int32)
    ang = _run(jflat, iidx, dflat, xyzflat, alflat, emb_flat)
    return atom_i_idx.reshape(-1), ang.reshape(N_CENTER, NP_, OUTW)
